# Initial kernel scaffold; baseline (speedup 1.0000x reference)
import jax
import jax.numpy as jnp
from jax.experimental import pallas as pl


def kernel(x, length, emb, W, b):
    raise NotImplementedError("write your pallas kernel here")


# speedup vs baseline: 6.0862x; 6.0862x over previous
import functools

import jax
import jax.numpy as jnp
from jax import lax
from jax.experimental import pallas as pl
from jax.experimental.pallas import tpu as pltpu
from jax.experimental.pallas import tpu_sc as plsc

_NC = 2
_NS = 16
_NW = _NC * _NS
_GRP = 128


@functools.partial(jax.jit, static_argnums=(3, 4))
def _sc_pool(emb, idx2d, tgt2d, B, DIM):
    groups_w = idx2d.shape[0] // _NW
    bags_w = B // _NW
    bags_sc = B // _NC
    mesh = plsc.VectorSubcoreMesh(core_axis_name="c", subcore_axis_name="s")

    @functools.partial(
        pl.kernel,
        mesh=mesh,
        out_type=jax.ShapeDtypeStruct((B, DIM), jnp.float32),
        scratch_types=[
            pltpu.VMEM((groups_w, _GRP), jnp.int32),
            pltpu.VMEM((groups_w, _GRP), jnp.int32),
            pltpu.VMEM((_GRP, DIM), jnp.float32),
            pltpu.VMEM_SHARED((bags_sc, DIM), jnp.float32),
        ],
    )
    def pool(emb_hbm, idx_hbm, tgt_hbm, out_hbm, idx_v, tgt_v, rows_v, shared_v):
        c = lax.axis_index("c")
        s = lax.axis_index("s")
        wid = c * _NS + s

        zeros = jnp.zeros((16,), jnp.float32)

        @pl.loop(0, _GRP)
        def _(r):
            @pl.loop(0, DIM, step=16)
            def _(c0):
                rows_v[r, pl.ds(c0, 16)] = zeros

        @pl.loop(0, bags_w, step=_GRP)
        def _(r0):
            pltpu.sync_copy(rows_v, shared_v.at[pl.ds(s * bags_w + r0, _GRP)])

        pltpu.sync_copy(idx_hbm.at[pl.ds(wid * groups_w, groups_w)], idx_v)
        pltpu.sync_copy(tgt_hbm, tgt_v)

        offv = jnp.full((16,), s * bags_w, jnp.int32)

        @pl.loop(0, groups_w)
        def _(g):
            @pl.loop(0, _GRP, step=16)
            def _(c0):
                tgt_v[g, pl.ds(c0, 16)] = tgt_v[g, pl.ds(c0, 16)] + offv

        @pl.loop(0, groups_w)
        def _(g):
            pltpu.sync_copy(emb_hbm.at[idx_v.at[g]], rows_v)
            pltpu.sync_copy(rows_v, shared_v.at[tgt_v.at[g]], add=True)

        pltpu.sync_copy(
            shared_v.at[pl.ds(s * bags_w, bags_w)],
            out_hbm.at[pl.ds(wid * bags_w, bags_w)],
        )

    return pool(emb, idx2d, tgt2d)


def _tc_project(pooled, length2d, W, b2d):
    B, DIM = pooled.shape
    HID = W.shape[0]
    BLK = 2048

    def body(p_ref, l_ref, w_ref, b_ref, o_ref):
        recip = 1.0 / l_ref[...].astype(jnp.float32)
        s = p_ref[...] * recip
        o_ref[...] = lax.dot_general(
            s, w_ref[...], (((1,), (1,)), ((), ())),
            preferred_element_type=jnp.float32,
        ) + b_ref[...]

    return pl.pallas_call(
        body,
        grid=(B // BLK,),
        in_specs=[
            pl.BlockSpec((BLK, DIM), lambda i: (i, 0)),
            pl.BlockSpec((BLK, 1), lambda i: (i, 0)),
            pl.BlockSpec((HID, DIM), lambda i: (0, 0)),
            pl.BlockSpec((1, HID), lambda i: (0, 0)),
        ],
        out_specs=pl.BlockSpec((BLK, HID), lambda i: (i, 0)),
        out_shape=jax.ShapeDtypeStruct((B, HID), jnp.float32),
    )(pooled, length2d, W, b2d)


def kernel(x, length, emb, W, b):
    B, L = x.shape
    DIM = emb.shape[1]
    HID = W.shape[0]

    idx2d = x.astype(jnp.int32).reshape(B * L // _GRP, _GRP)
    tgt2d = (jnp.arange(B * L // _NW, dtype=jnp.int32) // L).reshape(-1, _GRP)

    pooled = _sc_pool(emb, idx2d, tgt2d, B, DIM)
    return _tc_project(pooled, length.reshape(B, 1), W, b.reshape(1, HID))


# speedup vs baseline: 8.3851x; 1.3777x over previous
import functools

import jax
import jax.numpy as jnp
from jax import lax
from jax.experimental import pallas as pl
from jax.experimental.pallas import tpu as pltpu
from jax.experimental.pallas import tpu_sc as plsc

_NC = 2
_NS = 16
_NW = _NC * _NS
_GRP = 128


@functools.partial(jax.jit, static_argnums=(3, 4))
def _sc_pool(emb, idx2d, tgt2d, B, DIM):
    groups_w = idx2d.shape[0] // _NW
    bags_w = B // _NW
    bags_sc = B // _NC
    mesh = plsc.VectorSubcoreMesh(core_axis_name="c", subcore_axis_name="s")

    nbuf = 2

    @functools.partial(
        pl.kernel,
        mesh=mesh,
        out_type=jax.ShapeDtypeStruct((B, DIM), jnp.float32),
        scratch_types=[
            pltpu.VMEM((groups_w, _GRP), jnp.int32),
            pltpu.VMEM((groups_w, _GRP), jnp.int32),
            pltpu.VMEM((nbuf, _GRP, DIM), jnp.float32),
            pltpu.VMEM_SHARED((bags_sc, DIM), jnp.float32),
            pltpu.SemaphoreType.DMA((nbuf,)),
            pltpu.SemaphoreType.DMA((nbuf,)),
        ],
    )
    def pool(emb_hbm, idx_hbm, tgt_hbm, out_hbm, idx_v, tgt_v, rows_v,
             shared_v, gsem, ssem):
        c = lax.axis_index("c")
        s = lax.axis_index("s")
        wid = c * _NS + s

        zeros = jnp.zeros((16,), jnp.float32)

        @pl.loop(0, _GRP)
        def _(r):
            @pl.loop(0, DIM, step=16)
            def _(c0):
                rows_v[0, r, pl.ds(c0, 16)] = zeros

        @pl.loop(0, bags_w, step=_GRP)
        def _(r0):
            pltpu.sync_copy(rows_v.at[0], shared_v.at[pl.ds(s * bags_w + r0, _GRP)])

        pltpu.sync_copy(idx_hbm.at[pl.ds(wid * groups_w, groups_w)], idx_v)
        pltpu.sync_copy(tgt_hbm, tgt_v)

        offv = jnp.full((16,), s * bags_w, jnp.int32)

        @pl.loop(0, groups_w)
        def _(g):
            @pl.loop(0, _GRP, step=16)
            def _(c0):
                tgt_v[g, pl.ds(c0, 16)] = tgt_v[g, pl.ds(c0, 16)] + offv

        def start_gather(g, b):
            pltpu.async_copy(emb_hbm.at[idx_v.at[g]], rows_v.at[b], gsem.at[b])

        def wait_gather(g, b):
            pltpu.make_async_copy(
                emb_hbm.at[idx_v.at[g]], rows_v.at[b], gsem.at[b]).wait()

        for b in range(nbuf):
            start_gather(b, b)

        @pl.loop(0, groups_w, step=nbuf)
        def _(g0):
            for b in range(nbuf):
                g = g0 + b
                wait_gather(g, b)
                pltpu.async_copy(
                    rows_v.at[b], shared_v.at[tgt_v.at[g]], ssem.at[b],
                    add=True)
                pltpu.make_async_copy(
                    rows_v.at[b], shared_v.at[tgt_v.at[g]], ssem.at[b]).wait()
                nxt = g + nbuf

                @pl.when(nxt < groups_w)
                def _():
                    start_gather(nxt, b)

        pltpu.sync_copy(
            shared_v.at[pl.ds(s * bags_w, bags_w)],
            out_hbm.at[pl.ds(wid * bags_w, bags_w)],
        )

    return pool(emb, idx2d, tgt2d)


def _tc_project(pooled, length2d, W, b2d):
    B, DIM = pooled.shape
    HID = W.shape[0]
    BLK = 2048

    def body(p_ref, l_ref, w_ref, b_ref, o_ref):
        recip = 1.0 / l_ref[...].astype(jnp.float32)
        s = p_ref[...] * recip
        o_ref[...] = lax.dot_general(
            s, w_ref[...], (((1,), (1,)), ((), ())),
            preferred_element_type=jnp.float32,
        ) + b_ref[...]

    return pl.pallas_call(
        body,
        grid=(B // BLK,),
        in_specs=[
            pl.BlockSpec((BLK, DIM), lambda i: (i, 0)),
            pl.BlockSpec((BLK, 1), lambda i: (i, 0)),
            pl.BlockSpec((HID, DIM), lambda i: (0, 0)),
            pl.BlockSpec((1, HID), lambda i: (0, 0)),
        ],
        out_specs=pl.BlockSpec((BLK, HID), lambda i: (i, 0)),
        out_shape=jax.ShapeDtypeStruct((B, HID), jnp.float32),
    )(pooled, length2d, W, b2d)


def kernel(x, length, emb, W, b):
    B, L = x.shape
    DIM = emb.shape[1]
    HID = W.shape[0]

    idx2d = x.astype(jnp.int32).reshape(B * L // _GRP, _GRP)
    tgt2d = (jnp.arange(B * L // _NW, dtype=jnp.int32) // L).reshape(-1, _GRP)

    pooled = _sc_pool(emb, idx2d, tgt2d, B, DIM)
    return _tc_project(pooled, length.reshape(B, 1), W, b.reshape(1, HID))
